# baseline (speedup 1.0000x reference)
import jax
import jax.numpy as jnp
from jax import lax
from jax.experimental import pallas as pl
from jax.experimental.pallas import tpu as pltpu
from jax.experimental.pallas import tpu_sc as plsc

N_USERS = 20000
N_ITEMS = 30000
N = N_USERS + N_ITEMS
E = 800000
D = 64
B = 4096

LANES = 16
CW = 32
NP = D // CW
EC = 128
SGE = 256
SROWS = 3200
NTEC = 16
GROUP = 4
RPT = SROWS // NTEC
NGRP = RPT // GROUP
DRAIN = 200
NCHUNK = N // DRAIN
KMAX = -(-NCHUNK // NTEC)
BU = B // EC
BPT = BU // NTEC

_f32 = jnp.float32
_i32 = jnp.int32


def _lane_bcast(v16, t):
    dn = lax.GatherDimensionNumbers(
        offset_dims=(), collapsed_slice_dims=(0,), start_index_map=(0,))
    return lax.gather(v16, jnp.full((LANES, 1), t, _i32), dn, (1,),
                      mode=lax.GatherScatterMode.PROMISE_IN_BOUNDS)


def _body(src_h, dst_h, w_h, user_h, item_h, t0,
          out_u, out_i, t1, t2, t3,
          acc, src_v, dst_v, w_v, rows_a, rows_b, zbuf, uv,
          sem_st, sem_a, sem_b, sem_sa, sem_sb):
    c = lax.axis_index("core")
    s = lax.axis_index("subcore")
    tables = (t0, t1, t2, t3)
    boff = c * N

    @pl.loop(0, DRAIN)
    def _(i):
        zbuf[i, pl.ds(0, LANES)] = jnp.zeros((LANES,), _f32)
        zbuf[i, pl.ds(LANES, LANES)] = jnp.zeros((LANES,), _f32)

    for k in range(KMAX):
        cid = s + k * NTEC
        @pl.when(cid < NCHUNK)
        def _():
            pltpu.async_copy(zbuf, acc.at[pl.ds(cid * DRAIN, DRAIN), :],
                             sem_st)
    for k in range(KMAX):
        cid = s + k * NTEC
        @pl.when(cid < NCHUNK)
        def _():
            pltpu.make_async_copy(zbuf, acc.at[pl.ds(cid * DRAIN, DRAIN), :],
                                  sem_st).wait()
    plsc.subcore_barrier()

    row0 = s * RPT
    rbufs = (rows_a, rows_b)
    rsems = (sem_a, sem_b)
    ssems = (sem_sa, sem_sb)

    def stage_start(grp):
        base = row0 + grp * GROUP
        pltpu.async_copy(src_h.at[pl.ds(base, GROUP), :], src_v, sem_st)
        pltpu.async_copy(dst_h.at[pl.ds(base, GROUP), :], dst_v, sem_st)
        pltpu.async_copy(w_h.at[pl.ds(base, GROUP), :], w_v, sem_st)

    def stage_wait(grp):
        base = row0 + grp * GROUP
        pltpu.make_async_copy(src_h.at[pl.ds(base, GROUP), :], src_v,
                              sem_st).wait()
        pltpu.make_async_copy(dst_h.at[pl.ds(base, GROUP), :], dst_v,
                              sem_st).wait()
        pltpu.make_async_copy(w_h.at[pl.ds(base, GROUP), :], w_v,
                              sem_st).wait()

    for layer in range(3):
        tin = tables[layer]
        tout = tables[layer + 1]

        def gather_start(i):
            return pltpu.async_copy(
                tin.at[pl.ds(boff, N)].at[src_v.at[i]],
                rbufs[i % 2], rsems[i % 2])

        def scatter_start(i):
            return pltpu.async_copy(
                rbufs[i % 2], acc.at[dst_v.at[i]], ssems[i % 2], add=True)

        stage_start(0)

        @pl.loop(0, NGRP)
        def _(grp):
            stage_wait(grp)
            gh = [gather_start(0)]
            sh = [None] * GROUP
            for i in range(GROUP):
                if i + 1 < GROUP:
                    if i >= 1:
                        sh[i - 1].wait()
                    gh.append(gather_start(i + 1))
                gh[i].wait()
                buf = rbufs[i % 2]

                @pl.loop(0, SGE, step=LANES)
                def _(eb):
                    w16 = w_v[i, pl.ds(eb, LANES)]
                    for t in range(LANES):
                        wb = _lane_bcast(w16, t)
                        e = eb + t
                        buf[e, pl.ds(0, LANES)] = buf[e, pl.ds(0, LANES)] * wb
                        buf[e, pl.ds(LANES, LANES)] = (
                            buf[e, pl.ds(LANES, LANES)] * wb)

                sh[i] = scatter_start(i)
            sh[GROUP - 2].wait()
            sh[GROUP - 1].wait()

            @pl.when(grp + 1 < NGRP)
            def _():
                stage_start(grp + 1)

        plsc.subcore_barrier()

        for k in range(KMAX):
            cid = s + k * NTEC
            @pl.when(cid < NCHUNK)
            def _():
                r0 = cid * DRAIN
                pltpu.async_copy(acc.at[pl.ds(r0, DRAIN), :],
                                 tout.at[pl.ds(boff + r0, DRAIN), :], sem_st)
        for k in range(KMAX):
            cid = s + k * NTEC
            @pl.when(cid < NCHUNK)
            def _():
                r0 = cid * DRAIN
                pltpu.make_async_copy(
                    acc.at[pl.ds(r0, DRAIN), :],
                    tout.at[pl.ds(boff + r0, DRAIN), :], sem_st).wait()
        for k in range(KMAX):
            cid = s + k * NTEC
            @pl.when(cid < NCHUNK)
            def _():
                pltpu.sync_copy(zbuf, acc.at[pl.ds(cid * DRAIN, DRAIN), :])
        plsc.subcore_barrier()

    gbufs = (rows_a.at[pl.ds(0, EC), :], rows_a.at[pl.ds(EC, EC), :],
             rows_b.at[pl.ds(0, EC), :], rows_b.at[pl.ds(EC, EC), :])
    for qidx, qoff, qout in ((user_h, 0, out_u), (item_h, N_USERS, out_i)):
        pltpu.sync_copy(qidx.at[s], uv)
        for jj in range(BPT):
            j = s * BPT + jj
            fh = [pltpu.async_copy(
                      tbl.at[pl.ds(boff + qoff, N)].at[uv.at[jj]], gb, sem_st)
                  for tbl, gb in zip(tables, gbufs)]
            for h in fh:
                h.wait()

            @pl.loop(0, EC)
            def _(e):
                for half in range(2):
                    sl = pl.ds(half * LANES, LANES)
                    gbufs[0][e, sl] = (gbufs[0][e, sl] + gbufs[1][e, sl] +
                                       gbufs[2][e, sl] + gbufs[3][e, sl]
                                       ) * 0.25

            pltpu.sync_copy(gbufs[0], qout.at[c, pl.ds(j * EC, EC), :])


def _make_kernel():
    mesh = plsc.VectorSubcoreMesh(core_axis_name="core", subcore_axis_name="subcore")
    out_type = (
        jax.ShapeDtypeStruct((NP, B, CW), _f32),
        jax.ShapeDtypeStruct((NP, B, CW), _f32),
        jax.ShapeDtypeStruct((NP * N, CW), _f32),
        jax.ShapeDtypeStruct((NP * N, CW), _f32),
        jax.ShapeDtypeStruct((NP * N, CW), _f32),
    )
    scratch = [
        pltpu.VMEM_SHARED((N, CW), _f32),
        pltpu.VMEM((GROUP, SGE), _i32),
        pltpu.VMEM((GROUP, SGE), _i32),
        pltpu.VMEM((GROUP, SGE), _f32),
        pltpu.VMEM((SGE, CW), _f32),
        pltpu.VMEM((SGE, CW), _f32),
        pltpu.VMEM((DRAIN, CW), _f32),
        pltpu.VMEM((BPT, EC), _i32),
        pltpu.SemaphoreType.DMA,
        pltpu.SemaphoreType.DMA,
        pltpu.SemaphoreType.DMA,
        pltpu.SemaphoreType.DMA,
        pltpu.SemaphoreType.DMA,
    ]
    return pl.kernel(_body, out_type=out_type, mesh=mesh, scratch_types=scratch,
                     compiler_params=pltpu.CompilerParams(
                         use_tc_tiling_on_sc=False))


_lgcn = _make_kernel()


@jax.jit
def kernel(user_emb, item_emb, edge_weight, edge_index, user, item):
    ego0 = jnp.concatenate([user_emb, item_emb], axis=0)
    t0 = ego0.reshape(N, NP, CW).transpose(1, 0, 2).reshape(NP * N, CW)
    padn = SROWS * SGE - E
    src = jnp.concatenate([edge_index[0], jnp.zeros((padn,), _i32)]
                          ).reshape(SROWS, SGE)
    dst = jnp.concatenate([edge_index[1], jnp.zeros((padn,), _i32)]
                          ).reshape(SROWS, SGE)
    w = jnp.concatenate([edge_weight, jnp.zeros((padn,), _f32)]
                        ).reshape(SROWS, SGE)
    user2 = user.reshape(NTEC, BPT, EC)
    item2 = item.reshape(NTEC, BPT, EC)
    out_u, out_i, _, _, _ = _lgcn(src, dst, w, user2, item2, t0)
    u = out_u.transpose(1, 0, 2).reshape(B, D)
    it = out_i.transpose(1, 0, 2).reshape(B, D)
    return u, it


# speedup vs baseline: 1.3642x; 1.3642x over previous
import jax
import jax.numpy as jnp
from jax import lax
from jax.experimental import pallas as pl
from jax.experimental.pallas import tpu as pltpu
from jax.experimental.pallas import tpu_sc as plsc

N_USERS = 20000
N_ITEMS = 30000
N = N_USERS + N_ITEMS
E = 800000
D = 64
B = 4096

LANES = 16
NB = D // LANES
EC = 128
SGE = 256
SROWS = 3200
NTEC = 16
GROUP = 5
RPT = SROWS // NTEC
NGRP = RPT // GROUP
NRING = 5
LOOKAHEAD = 3
DRAIN = 200
NCHUNK = N // DRAIN
KMAX = -(-NCHUNK // NTEC)
BU = B // EC
BPT = BU // NTEC

_f32 = jnp.float32
_i32 = jnp.int32


def _lane_bcast(v16, t):
    dn = lax.GatherDimensionNumbers(
        offset_dims=(), collapsed_slice_dims=(0,), start_index_map=(0,))
    return lax.gather(v16, jnp.full((LANES, 1), t, _i32), dn, (1,),
                      mode=lax.GatherScatterMode.PROMISE_IN_BOUNDS)


def _body(src_h, dst_h, w_h, user_h, item_h, t0,
          out_u, out_i, t1, t2, t3,
          acc, src_v, dst_v, w_v, rows_r, zbuf, uv,
          sem_st, sem_a, sem_b, sem_c, sem_d, sem_e,
          sem_sa, sem_sb, sem_sc, sem_sd, sem_se):
    c = lax.axis_index("core")
    s = lax.axis_index("subcore")
    tables = (t0, t1, t2, t3)

    @pl.loop(0, DRAIN)
    def _(i):
        zbuf[i, :] = jnp.zeros((LANES,), _f32)

    zh = []
    for tb in range(2):
        for k in range(KMAX):
            cid = s + k * NTEC
            @pl.when(cid < NCHUNK)
            def _():
                zh.append(pltpu.async_copy(
                    zbuf, acc.at[tb, pl.ds(cid * DRAIN, DRAIN), :], sem_st))
    for tb in range(2):
        for k in range(KMAX):
            cid = s + k * NTEC
            @pl.when(cid < NCHUNK)
            def _():
                pltpu.make_async_copy(
                    zbuf, acc.at[tb, pl.ds(cid * DRAIN, DRAIN), :],
                    sem_st).wait()
    plsc.subcore_barrier()

    row0 = s * RPT
    rbufs = tuple(rows_r.at[pl.ds(k * SGE, SGE), :] for k in range(NRING))
    rsems = (sem_a, sem_b, sem_c, sem_d, sem_e)
    ssems = (sem_sa, sem_sb, sem_sc, sem_sd, sem_se)

    def stage_start(grp):
        base = row0 + grp * GROUP
        pltpu.async_copy(src_h.at[pl.ds(base, GROUP), :], src_v, sem_st)
        pltpu.async_copy(dst_h.at[pl.ds(base, GROUP), :], dst_v, sem_st)
        pltpu.async_copy(w_h.at[pl.ds(base, GROUP), :], w_v, sem_st)

    def stage_wait(grp):
        base = row0 + grp * GROUP
        pltpu.make_async_copy(src_h.at[pl.ds(base, GROUP), :], src_v,
                              sem_st).wait()
        pltpu.make_async_copy(dst_h.at[pl.ds(base, GROUP), :], dst_v,
                              sem_st).wait()
        pltpu.make_async_copy(w_h.at[pl.ds(base, GROUP), :], w_v,
                              sem_st).wait()

    NU = 2 * GROUP

    for layer in range(3):
        tin = tables[layer]
        tout = tables[layer + 1]

        def gather_start(i):
            r, tb = i // 2, i % 2
            boff = (2 * c + tb) * N
            return pltpu.async_copy(
                tin.at[pl.ds(boff, N)].at[src_v.at[r]],
                rbufs[i % NRING], rsems[i % NRING])

        def scatter_start(i):
            r, tb = i // 2, i % 2
            return pltpu.async_copy(
                rbufs[i % NRING], acc.at[tb].at[dst_v.at[r]],
                ssems[i % NRING], add=True)

        stage_start(0)

        @pl.loop(0, NGRP)
        def _(grp):
            stage_wait(grp)
            gh = [gather_start(k) for k in range(LOOKAHEAD)]
            sh = [None] * NU
            for i in range(NU):
                j = i + LOOKAHEAD
                if j < NU:
                    if j >= NRING:
                        sh[j - NRING].wait()
                    gh.append(gather_start(j))
                gh[i].wait()
                r, tb = i // 2, i % 2
                buf = rbufs[i % NRING]

                @pl.loop(0, SGE, step=LANES)
                def _(eb):
                    w16 = w_v[r, pl.ds(eb, LANES)]
                    for t in range(LANES):
                        wb = _lane_bcast(w16, t)
                        buf[eb + t, :] = buf[eb + t, :] * wb

                sh[i] = scatter_start(i)
            for i in range(NU - NRING, NU):
                sh[i].wait()

            @pl.when(grp + 1 < NGRP)
            def _():
                stage_start(grp + 1)

        plsc.subcore_barrier()

        for tb in range(2):
            boff = (2 * c + tb) * N
            for k in range(KMAX):
                cid = s + k * NTEC
                @pl.when(cid < NCHUNK)
                def _():
                    r0 = cid * DRAIN
                    pltpu.async_copy(acc.at[tb, pl.ds(r0, DRAIN), :],
                                     tout.at[pl.ds(boff + r0, DRAIN), :],
                                     sem_st)
        for tb in range(2):
            boff = (2 * c + tb) * N
            for k in range(KMAX):
                cid = s + k * NTEC
                @pl.when(cid < NCHUNK)
                def _():
                    r0 = cid * DRAIN
                    pltpu.make_async_copy(
                        acc.at[tb, pl.ds(r0, DRAIN), :],
                        tout.at[pl.ds(boff + r0, DRAIN), :], sem_st).wait()
        for tb in range(2):
            for k in range(KMAX):
                cid = s + k * NTEC
                @pl.when(cid < NCHUNK)
                def _():
                    pltpu.sync_copy(zbuf,
                                    acc.at[tb, pl.ds(cid * DRAIN, DRAIN), :])
        plsc.subcore_barrier()

    gbufs = [rows_r.at[pl.ds(k * EC, EC), :] for k in range(4)]
    obuf = rows_r.at[pl.ds(4 * EC, EC), :]
    for qidx, qoff, qout in ((user_h, 0, out_u), (item_h, N_USERS, out_i)):
        pltpu.sync_copy(qidx.at[s], uv)
        for jj in range(BPT):
            j = s * BPT + jj
            for tb in range(2):
                t = 2 * c + tb
                boff = t * N + qoff
                fh = [pltpu.async_copy(tbl.at[pl.ds(boff, N)].at[uv.at[jj]],
                                       gb, sem_st)
                      for tbl, gb in zip(tables, gbufs)]
                for h in fh:
                    h.wait()

                @pl.loop(0, EC)
                def _(e):
                    obuf[e, :] = (gbufs[0][e, :] + gbufs[1][e, :] +
                                  gbufs[2][e, :] + gbufs[3][e, :]) * 0.25

                pltpu.sync_copy(obuf, qout.at[t, pl.ds(j * EC, EC), :])


def _make_kernel():
    mesh = plsc.VectorSubcoreMesh(core_axis_name="core", subcore_axis_name="subcore")
    out_type = (
        jax.ShapeDtypeStruct((NB, B, LANES), _f32),
        jax.ShapeDtypeStruct((NB, B, LANES), _f32),
        jax.ShapeDtypeStruct((NB * N, LANES), _f32),
        jax.ShapeDtypeStruct((NB * N, LANES), _f32),
        jax.ShapeDtypeStruct((NB * N, LANES), _f32),
    )
    scratch = [
        pltpu.VMEM_SHARED((2, N, LANES), _f32),
        pltpu.VMEM((GROUP, SGE), _i32),
        pltpu.VMEM((GROUP, SGE), _i32),
        pltpu.VMEM((GROUP, SGE), _f32),
        pltpu.VMEM((NRING * SGE, LANES), _f32),
        pltpu.VMEM((DRAIN, LANES), _f32),
        pltpu.VMEM((BPT, EC), _i32),
        pltpu.SemaphoreType.DMA,
    ] + [pltpu.SemaphoreType.DMA] * 10
    return pl.kernel(_body, out_type=out_type, mesh=mesh, scratch_types=scratch,
                     compiler_params=pltpu.CompilerParams(
                         use_tc_tiling_on_sc=False))


_lgcn = _make_kernel()


@jax.jit
def kernel(user_emb, item_emb, edge_weight, edge_index, user, item):
    ego0 = jnp.concatenate([user_emb, item_emb], axis=0)
    t0 = ego0.reshape(N, NB, LANES).transpose(1, 0, 2).reshape(NB * N, LANES)
    padn = SROWS * SGE - E
    src = jnp.concatenate([edge_index[0], jnp.zeros((padn,), _i32)]
                          ).reshape(SROWS, SGE)
    dst = jnp.concatenate([edge_index[1], jnp.zeros((padn,), _i32)]
                          ).reshape(SROWS, SGE)
    w = jnp.concatenate([edge_weight, jnp.zeros((padn,), _f32)]
                        ).reshape(SROWS, SGE)
    user2 = user.reshape(NTEC, BPT, EC)
    item2 = item.reshape(NTEC, BPT, EC)
    out_u, out_i, _, _, _ = _lgcn(src, dst, w, user2, item2, t0)
    u = out_u.transpose(1, 0, 2).reshape(B, D)
    it = out_i.transpose(1, 0, 2).reshape(B, D)
    return u, it


# speedup vs baseline: 1.4660x; 1.0747x over previous
import jax
import jax.numpy as jnp
from jax import lax
from jax.experimental import pallas as pl
from jax.experimental.pallas import tpu as pltpu
from jax.experimental.pallas import tpu_sc as plsc

N_USERS = 20000
N_ITEMS = 30000
N = N_USERS + N_ITEMS
E = 800000
D = 64
B = 4096

LANES = 16
NB = D // LANES
EC = 128
SGE = 224
SROWS = 3648
NTEC = 16
GROUP = 6
RPT = SROWS // NTEC
NGRP = RPT // GROUP
NRING = 6
LOOKAHEAD = 4
DRAIN = 200
NCHUNK = N // DRAIN
KMAX = -(-NCHUNK // NTEC)
BU = B // EC
BPT = BU // NTEC

_f32 = jnp.float32
_i32 = jnp.int32


def _lane_bcast(v16, t):
    dn = lax.GatherDimensionNumbers(
        offset_dims=(), collapsed_slice_dims=(0,), start_index_map=(0,))
    return lax.gather(v16, jnp.full((LANES, 1), t, _i32), dn, (1,),
                      mode=lax.GatherScatterMode.PROMISE_IN_BOUNDS)


def _body(src_h, dst_h, w_h, user_h, item_h, t0,
          out_u, out_i, t1, t2, t3,
          acc, src_v, dst_v, w_v, rows_r, zbuf, uv,
          sem_st, sem_a, sem_b, sem_c, sem_d, sem_e, sem_f,
          sem_sa, sem_sb, sem_sc, sem_sd, sem_se, sem_sf):
    c = lax.axis_index("core")
    s = lax.axis_index("subcore")
    tables = (t0, t1, t2, t3)

    @pl.loop(0, DRAIN)
    def _(i):
        zbuf[i, :] = jnp.zeros((LANES,), _f32)

    zh = []
    for tb in range(2):
        for k in range(KMAX):
            cid = s + k * NTEC
            @pl.when(cid < NCHUNK)
            def _():
                zh.append(pltpu.async_copy(
                    zbuf, acc.at[tb, pl.ds(cid * DRAIN, DRAIN), :], sem_st))
    for tb in range(2):
        for k in range(KMAX):
            cid = s + k * NTEC
            @pl.when(cid < NCHUNK)
            def _():
                pltpu.make_async_copy(
                    zbuf, acc.at[tb, pl.ds(cid * DRAIN, DRAIN), :],
                    sem_st).wait()
    plsc.subcore_barrier()

    row0 = s * RPT
    rbufs = tuple(rows_r.at[pl.ds(k * SGE, SGE), :] for k in range(NRING))
    rsems = (sem_a, sem_b, sem_c, sem_d, sem_e, sem_f)
    ssems = (sem_sa, sem_sb, sem_sc, sem_sd, sem_se, sem_sf)

    def stage_start(grp):
        base = row0 + grp * GROUP
        pltpu.async_copy(src_h.at[pl.ds(base, GROUP), :], src_v, sem_st)
        pltpu.async_copy(dst_h.at[pl.ds(base, GROUP), :], dst_v, sem_st)
        pltpu.async_copy(w_h.at[pl.ds(base, GROUP), :], w_v, sem_st)

    def stage_wait(grp):
        base = row0 + grp * GROUP
        pltpu.make_async_copy(src_h.at[pl.ds(base, GROUP), :], src_v,
                              sem_st).wait()
        pltpu.make_async_copy(dst_h.at[pl.ds(base, GROUP), :], dst_v,
                              sem_st).wait()
        pltpu.make_async_copy(w_h.at[pl.ds(base, GROUP), :], w_v,
                              sem_st).wait()

    NU = 2 * GROUP

    for layer in range(3):
        tin = tables[layer]
        tout = tables[layer + 1]

        def gather_start(i):
            r, tb = i // 2, i % 2
            boff = (2 * c + tb) * N
            return pltpu.async_copy(
                tin.at[pl.ds(boff, N)].at[src_v.at[r]],
                rbufs[i % NRING], rsems[i % NRING])

        def scatter_start(i):
            r, tb = i // 2, i % 2
            return pltpu.async_copy(
                rbufs[i % NRING], acc.at[tb].at[dst_v.at[r]],
                ssems[i % NRING], add=True)

        stage_start(0)

        @pl.loop(0, NGRP)
        def _(grp):
            stage_wait(grp)
            gh = [gather_start(k) for k in range(LOOKAHEAD)]
            sh = [None] * NU
            for i in range(NU):
                j = i + LOOKAHEAD
                if j < NU:
                    if j >= NRING:
                        sh[j - NRING].wait()
                    gh.append(gather_start(j))
                gh[i].wait()
                r, tb = i // 2, i % 2
                buf = rbufs[i % NRING]

                @pl.loop(0, SGE, step=LANES)
                def _(eb):
                    w16 = w_v[r, pl.ds(eb, LANES)]
                    for t in range(LANES):
                        wb = _lane_bcast(w16, t)
                        buf[eb + t, :] = buf[eb + t, :] * wb

                sh[i] = scatter_start(i)
            for i in range(NU - NRING, NU):
                sh[i].wait()

            @pl.when(grp + 1 < NGRP)
            def _():
                stage_start(grp + 1)

        plsc.subcore_barrier()

        for tb in range(2):
            boff = (2 * c + tb) * N
            for k in range(KMAX):
                cid = s + k * NTEC
                @pl.when(cid < NCHUNK)
                def _():
                    r0 = cid * DRAIN
                    pltpu.async_copy(acc.at[tb, pl.ds(r0, DRAIN), :],
                                     tout.at[pl.ds(boff + r0, DRAIN), :],
                                     sem_st)
        for tb in range(2):
            boff = (2 * c + tb) * N
            for k in range(KMAX):
                cid = s + k * NTEC
                @pl.when(cid < NCHUNK)
                def _():
                    r0 = cid * DRAIN
                    pltpu.make_async_copy(
                        acc.at[tb, pl.ds(r0, DRAIN), :],
                        tout.at[pl.ds(boff + r0, DRAIN), :], sem_st).wait()
        for tb in range(2):
            for k in range(KMAX):
                cid = s + k * NTEC
                @pl.when(cid < NCHUNK)
                def _():
                    pltpu.sync_copy(zbuf,
                                    acc.at[tb, pl.ds(cid * DRAIN, DRAIN), :])
        plsc.subcore_barrier()

    gbufs = [rows_r.at[pl.ds(k * EC, EC), :] for k in range(4)]
    obuf = rows_r.at[pl.ds(4 * EC, EC), :]
    for qidx, qoff, qout in ((user_h, 0, out_u), (item_h, N_USERS, out_i)):
        pltpu.sync_copy(qidx.at[s], uv)
        for jj in range(BPT):
            j = s * BPT + jj
            for tb in range(2):
                t = 2 * c + tb
                boff = t * N + qoff
                fh = [pltpu.async_copy(tbl.at[pl.ds(boff, N)].at[uv.at[jj]],
                                       gb, sem_st)
                      for tbl, gb in zip(tables, gbufs)]
                for h in fh:
                    h.wait()

                @pl.loop(0, EC)
                def _(e):
                    obuf[e, :] = (gbufs[0][e, :] + gbufs[1][e, :] +
                                  gbufs[2][e, :] + gbufs[3][e, :]) * 0.25

                pltpu.sync_copy(obuf, qout.at[t, pl.ds(j * EC, EC), :])


def _make_kernel():
    mesh = plsc.VectorSubcoreMesh(core_axis_name="core", subcore_axis_name="subcore")
    out_type = (
        jax.ShapeDtypeStruct((NB, B, LANES), _f32),
        jax.ShapeDtypeStruct((NB, B, LANES), _f32),
        jax.ShapeDtypeStruct((NB * N, LANES), _f32),
        jax.ShapeDtypeStruct((NB * N, LANES), _f32),
        jax.ShapeDtypeStruct((NB * N, LANES), _f32),
    )
    scratch = [
        pltpu.VMEM_SHARED((2, N, LANES), _f32),
        pltpu.VMEM((GROUP, SGE), _i32),
        pltpu.VMEM((GROUP, SGE), _i32),
        pltpu.VMEM((GROUP, SGE), _f32),
        pltpu.VMEM((NRING * SGE, LANES), _f32),
        pltpu.VMEM((DRAIN, LANES), _f32),
        pltpu.VMEM((BPT, EC), _i32),
        pltpu.SemaphoreType.DMA,
    ] + [pltpu.SemaphoreType.DMA] * 12
    return pl.kernel(_body, out_type=out_type, mesh=mesh, scratch_types=scratch,
                     compiler_params=pltpu.CompilerParams(
                         use_tc_tiling_on_sc=False))


_lgcn = _make_kernel()


@jax.jit
def kernel(user_emb, item_emb, edge_weight, edge_index, user, item):
    ego0 = jnp.concatenate([user_emb, item_emb], axis=0)
    t0 = ego0.reshape(N, NB, LANES).transpose(1, 0, 2).reshape(NB * N, LANES)
    padn = SROWS * SGE - E
    src = jnp.concatenate([edge_index[0], jnp.zeros((padn,), _i32)]
                          ).reshape(SROWS, SGE)
    dst = jnp.concatenate([edge_index[1], jnp.zeros((padn,), _i32)]
                          ).reshape(SROWS, SGE)
    w = jnp.concatenate([edge_weight, jnp.zeros((padn,), _f32)]
                        ).reshape(SROWS, SGE)
    user2 = user.reshape(NTEC, BPT, EC)
    item2 = item.reshape(NTEC, BPT, EC)
    out_u, out_i, _, _, _ = _lgcn(src, dst, w, user2, item2, t0)
    u = out_u.transpose(1, 0, 2).reshape(B, D)
    it = out_i.transpose(1, 0, 2).reshape(B, D)
    return u, it
